# baseline (speedup 1.0000x reference)
import functools

import jax
import jax.numpy as jnp
from jax import lax
from jax.experimental import pallas as pl
from jax.experimental.pallas import tpu as pltpu
from jax.experimental.pallas import tpu_sc as plsc

NUM_CORES = 2
NUM_SUBCORES = 16
B_CORE0 = 480
B_CORE1 = 544
B_PAIR = B_CORE0 + B_CORE1


def kernel(labels, embedding_table):
    (B,) = labels.shape
    V, D = embedding_table.shape

    labels_1d = labels.astype(jnp.int32)
    mesh = plsc.VectorSubcoreMesh(core_axis_name="c", subcore_axis_name="s")

    @functools.partial(
        pl.kernel,
        mesh=mesh,
        out_type=jax.ShapeDtypeStruct((B, D), jnp.float32),
        scratch_types=[
            pltpu.VMEM((max(B_CORE0, B_CORE1),), jnp.int32),
            pltpu.VMEM((max(B_CORE0, B_CORE1), D), jnp.float32),
            pltpu.SemaphoreType.DMA,
        ],
    )
    def emb(table_hbm, labels_hbm, out_hbm, idx_v, rows_v, sem):
        cid = lax.axis_index("c")
        sid = lax.axis_index("s")

        @pl.when(cid == 0)
        def _():
            base = sid * B_PAIR
            pltpu.sync_copy(
                labels_hbm.at[pl.ds(base, B_CORE0)], idx_v.at[pl.ds(0, B_CORE0)]
            )
            pltpu.async_copy(
                table_hbm.at[idx_v.at[pl.ds(0, B_CORE0)]],
                rows_v.at[pl.ds(0, B_CORE0)],
                sem,
            ).wait()
            pltpu.sync_copy(
                rows_v.at[pl.ds(0, B_CORE0)], out_hbm.at[pl.ds(base, B_CORE0)]
            )

        @pl.when(cid == 1)
        def _():
            base = sid * B_PAIR + B_CORE0
            pltpu.sync_copy(
                labels_hbm.at[pl.ds(base, B_CORE1)], idx_v.at[pl.ds(0, B_CORE1)]
            )
            pltpu.async_copy(
                table_hbm.at[idx_v.at[pl.ds(0, B_CORE1)]],
                rows_v.at[pl.ds(0, B_CORE1)],
                sem,
            ).wait()
            pltpu.sync_copy(
                rows_v.at[pl.ds(0, B_CORE1)], out_hbm.at[pl.ds(base, B_CORE1)]
            )

    return emb(embedding_table, labels_1d)


# speedup vs baseline: 1.2196x; 1.2196x over previous
import functools

import jax
import jax.numpy as jnp
from jax import lax
from jax.experimental import pallas as pl
from jax.experimental.pallas import tpu as pltpu
from jax.experimental.pallas import tpu_sc as plsc

NUM_CORES = 2
NUM_SUBCORES = 16
NW = NUM_CORES * NUM_SUBCORES
CHUNK = 128
VPAD = 1024
STG = VPAD // NUM_SUBCORES


def kernel(labels, embedding_table):
    (B,) = labels.shape
    V, D = embedding_table.shape
    b_per_w = B // NW
    n_ch = b_per_w // CHUNK

    labels_1d = labels.astype(jnp.int32)
    table_pad = jnp.pad(embedding_table, ((0, VPAD - V), (0, 0)))
    mesh = plsc.VectorSubcoreMesh(core_axis_name="c", subcore_axis_name="s")

    @functools.partial(
        pl.kernel,
        mesh=mesh,
        out_type=jax.ShapeDtypeStruct((B, D), jnp.float32),
        scratch_types=[
            pltpu.VMEM((b_per_w,), jnp.int32),
            pltpu.VMEM((b_per_w, D), jnp.float32),
            pltpu.VMEM_SHARED((VPAD, D), jnp.float32),
        ]
        + [pltpu.SemaphoreType.DMA] * n_ch
        + [pltpu.SemaphoreType.DMA],
    )
    def emb(table_hbm, labels_hbm, out_hbm, idx_v, rows_v, table_sp, *sems):
        gsems, osem = sems[:n_ch], sems[n_ch]
        cid = lax.axis_index("c")
        sid = lax.axis_index("s")
        wid = sid * NUM_CORES + cid
        base = wid * b_per_w

        pltpu.sync_copy(table_hbm.at[pl.ds(sid * STG, STG)], rows_v.at[pl.ds(0, STG)])
        pltpu.sync_copy(rows_v.at[pl.ds(0, STG)], table_sp.at[pl.ds(sid * STG, STG)])
        pltpu.sync_copy(labels_hbm.at[pl.ds(base, b_per_w)], idx_v)
        plsc.subcore_barrier()

        gathers = [None] * n_ch
        gathers[0] = pltpu.async_copy(
            table_sp.at[idx_v.at[pl.ds(0, CHUNK)]], rows_v.at[pl.ds(0, CHUNK)], gsems[0]
        )
        stores = []
        for j in range(n_ch):
            gathers[j].wait()
            if j + 1 < n_ch:
                gathers[j + 1] = pltpu.async_copy(
                    table_sp.at[idx_v.at[pl.ds((j + 1) * CHUNK, CHUNK)]],
                    rows_v.at[pl.ds((j + 1) * CHUNK, CHUNK)],
                    gsems[j + 1],
                )
            stores.append(
                pltpu.async_copy(
                    rows_v.at[pl.ds(j * CHUNK, CHUNK)],
                    out_hbm.at[pl.ds(base + j * CHUNK, CHUNK)],
                    osem,
                )
            )
        for c in stores:
            c.wait()

    return emb(table_pad, labels_1d)


# speedup vs baseline: 1.2289x; 1.0076x over previous
import functools

import jax
import jax.numpy as jnp
from jax import lax
from jax.experimental import pallas as pl
from jax.experimental.pallas import tpu as pltpu
from jax.experimental.pallas import tpu_sc as plsc

NUM_CORES = 2
NUM_SUBCORES = 16
NW = NUM_CORES * NUM_SUBCORES
CHUNK = 64
VPAD = 1024
STG = VPAD // NUM_SUBCORES


def kernel(labels, embedding_table):
    (B,) = labels.shape
    V, D = embedding_table.shape
    b_per_w = B // NW
    n_ch = b_per_w // CHUNK

    labels_1d = labels.astype(jnp.int32)
    table_pad = jnp.pad(embedding_table, ((0, VPAD - V), (0, 0)))
    mesh = plsc.VectorSubcoreMesh(core_axis_name="c", subcore_axis_name="s")

    @functools.partial(
        pl.kernel,
        mesh=mesh,
        out_type=jax.ShapeDtypeStruct((B, D), jnp.float32),
        scratch_types=[
            pltpu.VMEM((b_per_w,), jnp.int32),
            pltpu.VMEM((b_per_w, D), jnp.float32),
            pltpu.VMEM((STG, D), jnp.float32),
            pltpu.VMEM_SHARED((VPAD, D), jnp.float32),
        ]
        + [pltpu.SemaphoreType.DMA] * n_ch
        + [pltpu.SemaphoreType.DMA] * 2,
    )
    def emb(table_hbm, labels_hbm, out_hbm, idx_v, rows_v, stage_v, table_sp, *sems):
        gsems, osem, tsem = sems[:n_ch], sems[n_ch], sems[n_ch + 1]
        cid = lax.axis_index("c")
        sid = lax.axis_index("s")
        wid = sid * NUM_CORES + cid
        base = wid * b_per_w

        a_tb = pltpu.async_copy(table_hbm.at[pl.ds(sid * STG, STG)], stage_v, tsem)
        pltpu.sync_copy(labels_hbm.at[pl.ds(base, b_per_w)], idx_v)
        gathers = [None] * n_ch
        gathers[0] = pltpu.async_copy(
            table_hbm.at[idx_v.at[pl.ds(0, CHUNK)]], rows_v.at[pl.ds(0, CHUNK)], gsems[0]
        )
        a_tb.wait()
        pltpu.sync_copy(stage_v, table_sp.at[pl.ds(sid * STG, STG)])
        plsc.subcore_barrier()

        stores = []
        for j in range(n_ch):
            gathers[j].wait()
            if j + 1 < n_ch:
                gathers[j + 1] = pltpu.async_copy(
                    table_sp.at[idx_v.at[pl.ds((j + 1) * CHUNK, CHUNK)]],
                    rows_v.at[pl.ds((j + 1) * CHUNK, CHUNK)],
                    gsems[j + 1],
                )
            stores.append(
                pltpu.async_copy(
                    rows_v.at[pl.ds(j * CHUNK, CHUNK)],
                    out_hbm.at[pl.ds(base + j * CHUNK, CHUNK)],
                    osem,
                )
            )
        for c in stores:
            c.wait()

    return emb(table_pad, labels_1d)


# speedup vs baseline: 1.2456x; 1.0136x over previous
import functools

import jax
import jax.numpy as jnp
from jax import lax
from jax.experimental import pallas as pl
from jax.experimental.pallas import tpu as pltpu
from jax.experimental.pallas import tpu_sc as plsc

NUM_CORES = 2
NUM_SUBCORES = 16
NW = NUM_CORES * NUM_SUBCORES
CHUNK = 128
VPAD = 1024
STG = VPAD // NUM_SUBCORES


def kernel(labels, embedding_table):
    (B,) = labels.shape
    V, D = embedding_table.shape
    b_per_w = B // NW
    n_ch = b_per_w // CHUNK

    labels_1d = labels.astype(jnp.int32)
    table_pad = jnp.pad(embedding_table, ((0, VPAD - V), (0, 0)))
    mesh = plsc.VectorSubcoreMesh(core_axis_name="c", subcore_axis_name="s")

    @functools.partial(
        pl.kernel,
        mesh=mesh,
        out_type=jax.ShapeDtypeStruct((B, D), jnp.float32),
        scratch_types=[
            pltpu.VMEM((b_per_w,), jnp.int32),
            pltpu.VMEM((b_per_w, D), jnp.float32),
            pltpu.VMEM((STG, D), jnp.float32),
            pltpu.VMEM_SHARED((VPAD, D), jnp.float32),
        ]
        + [pltpu.SemaphoreType.DMA] * n_ch
        + [pltpu.SemaphoreType.DMA] * 2,
    )
    def emb(table_hbm, labels_hbm, out_hbm, idx_v, rows_v, stage_v, table_sp, *sems):
        gsems, osem, tsem = sems[:n_ch], sems[n_ch], sems[n_ch + 1]
        cid = lax.axis_index("c")
        sid = lax.axis_index("s")
        wid = sid * NUM_CORES + cid
        base = wid * b_per_w

        a_tb = pltpu.async_copy(table_hbm.at[pl.ds(sid * STG, STG)], stage_v, tsem)
        pltpu.sync_copy(labels_hbm.at[pl.ds(base, b_per_w)], idx_v)
        gathers = [None] * n_ch
        gathers[0] = pltpu.async_copy(
            table_hbm.at[idx_v.at[pl.ds(0, CHUNK)]], rows_v.at[pl.ds(0, CHUNK)], gsems[0]
        )
        a_tb.wait()
        pltpu.sync_copy(stage_v, table_sp.at[pl.ds(sid * STG, STG)])
        plsc.subcore_barrier()

        stores = []
        for j in range(n_ch):
            gathers[j].wait()
            if j + 1 < n_ch:
                gathers[j + 1] = pltpu.async_copy(
                    table_sp.at[idx_v.at[pl.ds((j + 1) * CHUNK, CHUNK)]],
                    rows_v.at[pl.ds((j + 1) * CHUNK, CHUNK)],
                    gsems[j + 1],
                )
            stores.append(
                pltpu.async_copy(
                    rows_v.at[pl.ds(j * CHUNK, CHUNK)],
                    out_hbm.at[pl.ds(base + j * CHUNK, CHUNK)],
                    osem,
                )
            )
        for c in stores:
            c.wait()

    return emb(table_pad, labels_1d)
